# Initial kernel scaffold; baseline (speedup 1.0000x reference)
import jax
import jax.numpy as jnp
from jax.experimental import pallas as pl


def kernel(coords, features):
    raise NotImplementedError("write your pallas kernel here")


# speedup vs baseline: 6.8487x; 6.8487x over previous
import functools

import jax
import jax.numpy as jnp
from jax import lax
from jax.experimental import pallas as pl
from jax.experimental.pallas import tpu as pltpu
from jax.experimental.pallas import tpu_sc as plsc

P0, P1, P2 = 73856093, 19349663, 83492791
TABLE = 1 << 20
N = 500000
D = 16

NW = 32
BPW = TABLE // NW
QB = 2048
VPB = QB // 16
CPT = 15632
HMASK = TABLE - 1

_mesh = plsc.VectorSubcoreMesh(core_axis_name="c", subcore_axis_name="s")
_params = pltpu.CompilerParams(
    needs_layout_passes=False, use_tc_tiling_on_sc=False
)


_I = jnp.int32


def _wid():
    return lax.axis_index("s") * _I(2) + lax.axis_index("c")


def _tile_range(wid):
    i0 = wid * _I(CPT)
    cnt = jnp.minimum(_I(CPT), _I(N) - i0)
    return i0, cnt


@functools.partial(
    pl.kernel,
    out_type=jax.ShapeDtypeStruct((N,), jnp.int32),
    mesh=_mesh,
    compiler_params=_params,
    scratch_types=[
        pltpu.VMEM((QB,), jnp.int32),
        pltpu.VMEM((QB,), jnp.int32),
        pltpu.VMEM((QB,), jnp.int32),
        pltpu.VMEM((QB,), jnp.int32),
    ],
)
def _hash_k(x_hbm, y_hbm, z_hbm, h_hbm, xv, yv, zv, hv):
    wid = _wid()
    i0, cnt = _tile_range(wid)
    trips = (cnt + _I(QB - 1)) // _I(QB)

    def block(k, carry):
        off = i0 + jnp.minimum(k * _I(QB), cnt - _I(QB))
        pltpu.sync_copy(x_hbm.at[pl.ds(off, QB)], xv)
        pltpu.sync_copy(y_hbm.at[pl.ds(off, QB)], yv)
        pltpu.sync_copy(z_hbm.at[pl.ds(off, QB)], zv)

        p0 = jnp.full((16,), P0, jnp.int32)
        p1 = jnp.full((16,), P1, jnp.int32)
        p2 = jnp.full((16,), P2, jnp.int32)
        hm = jnp.full((16,), HMASK, jnp.int32)

        def vec(v, c2):
            s = v * _I(16)
            h = (
                xv[pl.ds(s, 16)] * p0
                + yv[pl.ds(s, 16)] * p1
                + zv[pl.ds(s, 16)] * p2
            ) & hm
            hv[pl.ds(s, 16)] = h
            return c2

        lax.fori_loop(_I(0), _I(VPB), vec, _I(0))
        pltpu.sync_copy(hv, h_hbm.at[pl.ds(off, QB)])
        return carry

    lax.fori_loop(_I(0), trips, block, _I(0))


@functools.partial(
    pl.kernel,
    out_type=jax.ShapeDtypeStruct((TABLE,), jnp.int32),
    mesh=_mesh,
    compiler_params=_params,
    scratch_types=[
        pltpu.VMEM((QB,), jnp.int32),
        pltpu.VMEM((BPW,), jnp.int32),
    ],
)
def _build_k(h_hbm, win_hbm, hv, win_v):
    wid = _wid()
    base = wid * _I(BPW)
    zeros = jnp.zeros((16,), jnp.int32)

    def zinit(k, carry):
        win_v[pl.ds(k * _I(16), 16)] = zeros
        return carry

    lax.fori_loop(_I(0), _I(BPW // 16), zinit, _I(0))

    lanes = lax.iota(jnp.int32, 16)
    trips = _I((N + QB - 1) // QB)

    def block(k, carry):
        off = jnp.minimum(k * _I(QB), _I(N - QB))
        pltpu.sync_copy(h_hbm.at[pl.ds(off, QB)], hv)

        def vec(v, c2):
            s = v * _I(16)
            idx = hv[pl.ds(s, 16)] - base
            m = (idx >= _I(0)) & (idx < _I(BPW))
            idxc = jnp.where(m, idx, _I(0))
            jv = off + s + lanes
            plsc.store_scatter(win_v, [idxc], jv, mask=m)
            return c2

        lax.fori_loop(_I(0), _I(VPB), vec, _I(0))
        return carry

    lax.fori_loop(_I(0), trips, block, _I(0))
    pltpu.sync_copy(win_v, win_hbm.at[pl.ds(base, BPW)])


@functools.partial(
    pl.kernel,
    out_type=jax.ShapeDtypeStruct((N, D), jnp.float32),
    mesh=_mesh,
    compiler_params=_params,
    scratch_types=[
        pltpu.VMEM((QB,), jnp.int32),
        pltpu.VMEM((QB,), jnp.int32),
        pltpu.VMEM((QB, D), jnp.float32),
        pltpu.SemaphoreType.DMA,
    ],
)
def _query_k(h_hbm, win_hbm, feat_hbm, out_hbm, hv, gv, rows_v, sem):
    wid = _wid()
    i0, cnt = _tile_range(wid)
    trips = (cnt + _I(QB - 1)) // _I(QB)

    def block(k, carry):
        off = i0 + jnp.minimum(k * _I(QB), cnt - _I(QB))
        pltpu.sync_copy(h_hbm.at[pl.ds(off, QB)], hv)
        pltpu.async_copy(win_hbm.at[hv], gv, sem).wait()
        pltpu.async_copy(feat_hbm.at[gv], rows_v, sem).wait()
        pltpu.sync_copy(rows_v, out_hbm.at[pl.ds(off, QB)])
        return carry

    lax.fori_loop(_I(0), trips, block, _I(0))


def kernel(coords, features):
    c = coords.astype(jnp.int32)
    x, y, z = c[:, 0], c[:, 1], c[:, 2]
    h = _hash_k(x, y, z)
    win = _build_k(h)
    return _query_k(h, win, features)


# speedup vs baseline: 8.4710x; 1.2369x over previous
import functools

import jax
import jax.numpy as jnp
from jax import lax
from jax.experimental import pallas as pl
from jax.experimental.pallas import tpu as pltpu
from jax.experimental.pallas import tpu_sc as plsc

P0, P1, P2 = 73856093, 19349663, 83492791
TABLE = 1 << 20
N = 500000
D = 16

NW = 32
BPW = TABLE // NW
QB = 2048
VPB = QB // 16
CPT = 15632
HMASK = TABLE - 1

_mesh = plsc.VectorSubcoreMesh(core_axis_name="c", subcore_axis_name="s")
_params = pltpu.CompilerParams(
    needs_layout_passes=False, use_tc_tiling_on_sc=False
)


_I = jnp.int32


def _wid():
    return lax.axis_index("s") * _I(2) + lax.axis_index("c")


def _tile_range(wid):
    i0 = wid * _I(CPT)
    cnt = jnp.minimum(_I(CPT), _I(N) - i0)
    return i0, cnt


@functools.partial(
    pl.kernel,
    out_type=jax.ShapeDtypeStruct((N,), jnp.int32),
    mesh=_mesh,
    compiler_params=_params,
    scratch_types=[
        pltpu.VMEM((QB,), jnp.int32),
        pltpu.VMEM((QB,), jnp.int32),
        pltpu.VMEM((QB,), jnp.int32),
        pltpu.VMEM((QB,), jnp.int32),
    ],
)
def _hash_k(x_hbm, y_hbm, z_hbm, h_hbm, xv, yv, zv, hv):
    wid = _wid()
    i0, cnt = _tile_range(wid)
    trips = (cnt + _I(QB - 1)) // _I(QB)

    def block(k, carry):
        off = i0 + jnp.minimum(k * _I(QB), cnt - _I(QB))
        pltpu.sync_copy(x_hbm.at[pl.ds(off, QB)], xv)
        pltpu.sync_copy(y_hbm.at[pl.ds(off, QB)], yv)
        pltpu.sync_copy(z_hbm.at[pl.ds(off, QB)], zv)

        p0 = jnp.full((16,), P0, jnp.int32)
        p1 = jnp.full((16,), P1, jnp.int32)
        p2 = jnp.full((16,), P2, jnp.int32)
        hm = jnp.full((16,), HMASK, jnp.int32)

        def vec(v, c2):
            s = v * _I(16)
            h = (
                xv[pl.ds(s, 16)] * p0
                + yv[pl.ds(s, 16)] * p1
                + zv[pl.ds(s, 16)] * p2
            ) & hm
            hv[pl.ds(s, 16)] = h
            return c2

        lax.fori_loop(_I(0), _I(VPB), vec, _I(0))
        pltpu.sync_copy(hv, h_hbm.at[pl.ds(off, QB)])
        return carry

    lax.fori_loop(_I(0), trips, block, _I(0))


BQB = 8192
BTRIPS = -(-N // BQB)
UNROLL = 8


@functools.partial(
    pl.kernel,
    out_type=jax.ShapeDtypeStruct((TABLE,), jnp.int32),
    mesh=_mesh,
    compiler_params=_params,
    scratch_types=[
        pltpu.VMEM((BQB,), jnp.int32),
        pltpu.VMEM((BQB,), jnp.int32),
        pltpu.VMEM((BPW,), jnp.int32),
        pltpu.SemaphoreType.DMA,
        pltpu.SemaphoreType.DMA,
    ],
)
def _build_k(h_hbm, win_hbm, hv0, hv1, win_v, sem0, sem1):
    wid = _wid()
    base = wid * _I(BPW)
    zeros = jnp.zeros((16,), jnp.int32)

    def zinit(k, carry):
        win_v[pl.ds(k * _I(16), 16)] = zeros
        return carry

    lax.fori_loop(_I(0), _I(BPW // 16), zinit, _I(0))

    lanes = lax.iota(jnp.int32, 16)
    ubpw = jnp.full((16,), BPW, jnp.uint32)

    def off_of(k):
        return jnp.minimum(k * _I(BQB), _I(N - BQB))

    def start(k, buf, sem):
        pltpu.async_copy(h_hbm.at[pl.ds(off_of(k), BQB)], buf, sem)

    def wait(buf, sem):
        pltpu.make_async_copy(h_hbm.at[pl.ds(0, BQB)], buf, sem).wait()

    def compute(buf, off):
        def vec(v, jv):
            s = v * _I(16 * UNROLL)
            for u in range(UNROLL):
                idx = buf[pl.ds(s + u * 16, 16)] - base
                m = plsc.bitcast(idx, jnp.uint32) < ubpw
                idxc = jnp.where(m, idx, _I(0))
                plsc.store_scatter(win_v, [idxc], jv + _I(u * 16), mask=m)
            return jv + _I(16 * UNROLL)

        lax.fori_loop(_I(0), _I(BQB // (16 * UNROLL)), vec, off + lanes)

    start(_I(0), hv0, sem0)

    def pair(kk, carry):
        k0 = kk * _I(2)
        start(k0 + _I(1), hv1, sem1)
        wait(hv0, sem0)
        compute(hv0, off_of(k0))
        start(k0 + _I(2), hv0, sem0)
        wait(hv1, sem1)
        compute(hv1, off_of(k0 + _I(1)))
        return carry

    lax.fori_loop(_I(0), _I(BTRIPS // 2), pair, _I(0))
    wait(hv0, sem0)
    pltpu.sync_copy(win_v, win_hbm.at[pl.ds(base, BPW)])


@functools.partial(
    pl.kernel,
    out_type=jax.ShapeDtypeStruct((N, D), jnp.float32),
    mesh=_mesh,
    compiler_params=_params,
    scratch_types=[
        pltpu.VMEM((QB,), jnp.int32),
        pltpu.VMEM((QB,), jnp.int32),
        pltpu.VMEM((QB, D), jnp.float32),
        pltpu.SemaphoreType.DMA,
    ],
)
def _query_k(h_hbm, win_hbm, feat_hbm, out_hbm, hv, gv, rows_v, sem):
    wid = _wid()
    i0, cnt = _tile_range(wid)
    trips = (cnt + _I(QB - 1)) // _I(QB)

    def block(k, carry):
        off = i0 + jnp.minimum(k * _I(QB), cnt - _I(QB))
        pltpu.sync_copy(h_hbm.at[pl.ds(off, QB)], hv)
        pltpu.async_copy(win_hbm.at[hv], gv, sem).wait()
        pltpu.async_copy(feat_hbm.at[gv], rows_v, sem).wait()
        pltpu.sync_copy(rows_v, out_hbm.at[pl.ds(off, QB)])
        return carry

    lax.fori_loop(_I(0), trips, block, _I(0))


def kernel(coords, features):
    c = coords.astype(jnp.int32)
    x, y, z = c[:, 0], c[:, 1], c[:, 2]
    h = _hash_k(x, y, z)
    win = _build_k(h)
    return _query_k(h, win, features)
